# initial kernel scaffold (unmeasured)
import jax
import jax.numpy as jnp
from jax import lax
from jax.experimental import pallas as pl
from jax.experimental.pallas import tpu as pltpu

N_DEV = 16
M = 4096
N_OUT = 2048
CHUNK = M // N_DEV

_GELU_C = 0.7978845608028654


def _gelu(y):
    return 0.5 * y * (1.0 + jnp.tanh(_GELU_C * (y + 0.044715 * y * y * y)))


def kernel(x, w_mat):
    def body(x_ref, w_ref, out_ref, send_buf, recv_buf, send_sems, recv_sems,
             credit_sem):
        my = lax.axis_index("i")
        left = lax.rem(my + N_DEV - 1, N_DEV)
        right = lax.rem(my + 1, N_DEV)

        barrier_sem = pltpu.get_barrier_semaphore()
        for nbr in (left, right):
            pl.semaphore_signal(
                barrier_sem, inc=1,
                device_id=(nbr,), device_id_type=pl.DeviceIdType.MESH,
            )
        pl.semaphore_wait(barrier_sem, 2)

        out_ref[...] = jnp.dot(
            x_ref[...].astype(jnp.bfloat16),
            w_ref[...].astype(jnp.bfloat16),
            preferred_element_type=jnp.float32,
        )

        def rows(c):
            return pl.ds(c * CHUNK, CHUNK)

        def mod(v):
            return lax.rem(v + 2 * N_DEV, N_DEV)

        T_TOTAL = 2 * (N_DEV - 1)

        def ring_step(t, c_send, consume):
            slot = t % 2
            if t >= 2:
                pl.semaphore_wait(credit_sem, 1)
            send_buf[slot] = out_ref[rows(c_send), :].astype(jnp.bfloat16)
            rdma = pltpu.make_async_remote_copy(
                src_ref=send_buf.at[slot],
                dst_ref=recv_buf.at[slot],
                send_sem=send_sems.at[slot],
                recv_sem=recv_sems.at[slot],
                device_id=(right,),
                device_id_type=pl.DeviceIdType.MESH,
            )
            rdma.start()
            rdma.wait()
            consume(slot)
            if t + 2 < T_TOTAL:
                pl.semaphore_signal(
                    credit_sem, inc=1,
                    device_id=(left,), device_id_type=pl.DeviceIdType.MESH,
                )

        for s in range(N_DEV - 1):
            c_recv = mod(my - s - 1)

            def consume_rs(slot, c_recv=c_recv):
                out_ref[rows(c_recv), :] += recv_buf[slot].astype(jnp.float32)

            ring_step(s, mod(my - s), consume_rs)

        c_own = mod(my + 1)
        out_ref[rows(c_own), :] = _gelu(out_ref[rows(c_own), :])

        for h in range(N_DEV - 1):
            c_recv = mod(my - h)

            def consume_ag(slot, c_recv=c_recv):
                out_ref[rows(c_recv), :] = recv_buf[slot].astype(jnp.float32)

            ring_step(N_DEV - 1 + h, mod(my + 1 - h), consume_ag)

    return pl.pallas_call(
        body,
        out_shape=jax.ShapeDtypeStruct((M, N_OUT), jnp.float32),
        in_specs=[
            pl.BlockSpec(memory_space=pltpu.VMEM),
            pl.BlockSpec(memory_space=pltpu.VMEM),
        ],
        out_specs=pl.BlockSpec(memory_space=pltpu.VMEM),
        scratch_shapes=[
            pltpu.VMEM((2, CHUNK, N_OUT), jnp.bfloat16),
            pltpu.VMEM((2, CHUNK, N_OUT), jnp.bfloat16),
            pltpu.SemaphoreType.DMA((2,)),
            pltpu.SemaphoreType.DMA((2,)),
            pltpu.SemaphoreType.REGULAR,
        ],
        compiler_params=pltpu.CompilerParams(collective_id=0),
    )(x, w_mat)


# baseline (device time: 446046 ns/iter reference)
import jax
import jax.numpy as jnp
from jax import lax
from jax.experimental import pallas as pl
from jax.experimental.pallas import tpu as pltpu

N_DEV = 16
M = 4096
N_OUT = 2048
CHUNK = M // N_DEV

_GELU_C = 0.7978845608028654


def _gelu(y):
    return 0.5 * y * (1.0 + jnp.tanh(_GELU_C * (y + 0.044715 * y * y * y)))


def kernel(x, w_mat):
    def body(x_ref, w_ref, out_ref, send_buf, recv_buf, send_sems, recv_sems,
             credit_sem):
        my = lax.axis_index("i")
        left = lax.rem(my + N_DEV - 1, N_DEV)
        right = lax.rem(my + 1, N_DEV)

        barrier_sem = pltpu.get_barrier_semaphore()
        for nbr in (left, right):
            pl.semaphore_signal(
                barrier_sem, inc=1,
                device_id=(nbr,), device_id_type=pl.DeviceIdType.MESH,
            )
        pl.semaphore_wait(barrier_sem, 2)

        out_ref[...] = jnp.dot(
            x_ref[...].astype(jnp.bfloat16),
            w_ref[...].astype(jnp.bfloat16),
            preferred_element_type=jnp.float32,
        )

        def rows(c):
            return pl.ds(c * CHUNK, CHUNK)

        def mod(v):
            return lax.rem(v + 2 * N_DEV, N_DEV)

        T_TOTAL = 2 * (N_DEV - 1)

        def ring_step(t, c_send, consume):
            slot = t % 2
            if t >= 2:
                pl.semaphore_wait(credit_sem, 1)
            send_buf[slot] = out_ref[rows(c_send), :].astype(jnp.bfloat16)
            rdma = pltpu.make_async_remote_copy(
                src_ref=send_buf.at[slot],
                dst_ref=recv_buf.at[slot],
                send_sem=send_sems.at[slot],
                recv_sem=recv_sems.at[slot],
                device_id=(right,),
                device_id_type=pl.DeviceIdType.MESH,
            )
            rdma.start()
            rdma.wait()
            consume(slot)
            if t + 2 < T_TOTAL:
                pl.semaphore_signal(
                    credit_sem, inc=1,
                    device_id=(left,), device_id_type=pl.DeviceIdType.MESH,
                )

        for s in range(N_DEV - 1):
            c_recv = mod(my - s - 1)

            def consume_rs(slot, c_recv=c_recv):
                out_ref[rows(c_recv), :] += recv_buf[slot].astype(jnp.float32)

            ring_step(s, mod(my - s), consume_rs)

        c_own = mod(my + 1)
        out_ref[rows(c_own), :] = _gelu(out_ref[rows(c_own), :])

        for h in range(N_DEV - 1):
            c_recv = mod(my - h)

            def consume_ag(slot, c_recv=c_recv):
                out_ref[rows(c_recv), :] = recv_buf[slot].astype(jnp.float32)

            ring_step(N_DEV - 1 + h, mod(my + 1 - h), consume_ag)

    return pl.pallas_call(
        body,
        out_shape=jax.ShapeDtypeStruct((M, N_OUT), jnp.float32),
        in_specs=[
            pl.BlockSpec(memory_space=pltpu.VMEM),
            pl.BlockSpec(memory_space=pltpu.VMEM),
        ],
        out_specs=pl.BlockSpec(memory_space=pltpu.VMEM),
        scratch_shapes=[
            pltpu.VMEM((2, CHUNK, N_OUT), jnp.bfloat16),
            pltpu.VMEM((2, CHUNK, N_OUT), jnp.bfloat16),
            pltpu.SemaphoreType.DMA((2,)),
            pltpu.SemaphoreType.DMA((2,)),
            pltpu.SemaphoreType.REGULAR,
        ],
        compiler_params=pltpu.CompilerParams(
            collective_id=0,
            vmem_limit_bytes=100 * 1024 * 1024,
        ),
    )(x, w_mat)


# device time: 318789 ns/iter; 1.3992x vs baseline; 1.3992x over previous
import jax
import jax.numpy as jnp
from jax import lax
from jax.experimental import pallas as pl
from jax.experimental.pallas import tpu as pltpu

N_DEV = 16
M = 4096
N_OUT = 2048
CHUNK = M // N_DEV
HALF = CHUNK // 2

_GELU_C = 0.7978845608028654


def _gelu(y):
    return 0.5 * y * (1.0 + jnp.tanh(_GELU_C * (y + 0.044715 * y * y * y)))


def kernel(x, w_mat):
    def body(x_ref, w_ref, out_ref,
             send_r, recv_r, send_l, recv_l,
             send_sems_r, recv_sems_r, send_sems_l, recv_sems_l,
             credit_r, credit_l):
        my = lax.axis_index("i")
        left = lax.rem(my + N_DEV - 1, N_DEV)
        right = lax.rem(my + 1, N_DEV)

        barrier_sem = pltpu.get_barrier_semaphore()
        for nbr in (left, right):
            pl.semaphore_signal(
                barrier_sem, inc=1,
                device_id=(nbr,), device_id_type=pl.DeviceIdType.MESH,
            )
        pl.semaphore_wait(barrier_sem, 2)

        out_ref[...] = jnp.dot(
            x_ref[...].astype(jnp.bfloat16),
            w_ref[...].astype(jnp.bfloat16),
            preferred_element_type=jnp.float32,
        )

        def top(c):
            return pl.ds(c * CHUNK, HALF)

        def bot(c):
            return pl.ds(c * CHUNK + HALF, HALF)

        def mod(v):
            return lax.rem(v + 2 * N_DEV, N_DEV)

        T_TOTAL = 2 * (N_DEV - 1)

        def ring_step(t, c_send_r, c_send_l, consume):
            slot = t % 2
            if t >= 2:
                pl.semaphore_wait(credit_r, 1)
                pl.semaphore_wait(credit_l, 1)
            send_r[slot] = out_ref[top(c_send_r), :].astype(jnp.bfloat16)
            send_l[slot] = out_ref[bot(c_send_l), :].astype(jnp.bfloat16)
            rdma_r = pltpu.make_async_remote_copy(
                src_ref=send_r.at[slot], dst_ref=recv_r.at[slot],
                send_sem=send_sems_r.at[slot], recv_sem=recv_sems_r.at[slot],
                device_id=(right,), device_id_type=pl.DeviceIdType.MESH,
            )
            rdma_l = pltpu.make_async_remote_copy(
                src_ref=send_l.at[slot], dst_ref=recv_l.at[slot],
                send_sem=send_sems_l.at[slot], recv_sem=recv_sems_l.at[slot],
                device_id=(left,), device_id_type=pl.DeviceIdType.MESH,
            )
            rdma_r.start()
            rdma_l.start()
            rdma_r.wait()
            rdma_l.wait()
            consume(slot)
            if t + 2 < T_TOTAL:
                pl.semaphore_signal(
                    credit_r, inc=1,
                    device_id=(left,), device_id_type=pl.DeviceIdType.MESH,
                )
                pl.semaphore_signal(
                    credit_l, inc=1,
                    device_id=(right,), device_id_type=pl.DeviceIdType.MESH,
                )

        for s in range(N_DEV - 1):
            c_recv_r = mod(my - s - 1)
            c_recv_l = mod(my + s + 1)

            def consume_rs(slot, c_r=c_recv_r, c_l=c_recv_l):
                out_ref[top(c_r), :] += recv_r[slot].astype(jnp.float32)
                out_ref[bot(c_l), :] += recv_l[slot].astype(jnp.float32)

            ring_step(s, mod(my - s), mod(my + s), consume_rs)

        c_own_r = mod(my + 1)
        c_own_l = mod(my - 1)
        out_ref[top(c_own_r), :] = _gelu(out_ref[top(c_own_r), :])
        out_ref[bot(c_own_l), :] = _gelu(out_ref[bot(c_own_l), :])

        for h in range(N_DEV - 1):
            c_recv_r = mod(my - h)
            c_recv_l = mod(my + h)

            def consume_ag(slot, c_r=c_recv_r, c_l=c_recv_l):
                out_ref[top(c_r), :] = recv_r[slot].astype(jnp.float32)
                out_ref[bot(c_l), :] = recv_l[slot].astype(jnp.float32)

            ring_step(N_DEV - 1 + h, mod(my + 1 - h), mod(my - 1 + h),
                      consume_ag)

    return pl.pallas_call(
        body,
        out_shape=jax.ShapeDtypeStruct((M, N_OUT), jnp.float32),
        in_specs=[
            pl.BlockSpec(memory_space=pltpu.VMEM),
            pl.BlockSpec(memory_space=pltpu.VMEM),
        ],
        out_specs=pl.BlockSpec(memory_space=pltpu.VMEM),
        scratch_shapes=[
            pltpu.VMEM((2, HALF, N_OUT), jnp.bfloat16),
            pltpu.VMEM((2, HALF, N_OUT), jnp.bfloat16),
            pltpu.VMEM((2, HALF, N_OUT), jnp.bfloat16),
            pltpu.VMEM((2, HALF, N_OUT), jnp.bfloat16),
            pltpu.SemaphoreType.DMA((2,)),
            pltpu.SemaphoreType.DMA((2,)),
            pltpu.SemaphoreType.DMA((2,)),
            pltpu.SemaphoreType.DMA((2,)),
            pltpu.SemaphoreType.REGULAR,
            pltpu.SemaphoreType.REGULAR,
        ],
        compiler_params=pltpu.CompilerParams(
            collective_id=0,
            vmem_limit_bytes=100 * 1024 * 1024,
        ),
    )(x, w_mat)


# device time: 232547 ns/iter; 1.9181x vs baseline; 1.3709x over previous
import jax
import jax.numpy as jnp
from jax import lax
from jax.experimental import pallas as pl
from jax.experimental.pallas import tpu as pltpu

N_DEV = 16
M = 4096
N_OUT = 2048
CHUNK = M // N_DEV
HALF = CHUNK // 2
NCOL = N_OUT // 2

RS_STEPS = N_DEV - 1
T_TOTAL = 2 * (N_DEV - 1)

_GELU_C = 0.7978845608028654


def _gelu(y):
    return 0.5 * y * (1.0 + jnp.tanh(_GELU_C * (y + 0.044715 * y * y * y)))


def kernel(x, w_mat):
    def body(x_ref, w_ref, out_ref,
             send_r_a, recv_r_a, send_l_a, recv_l_a,
             send_r_b, recv_r_b, send_l_b, recv_l_b,
             ssr_a, rsr_a, ssl_a, rsl_a,
             ssr_b, rsr_b, ssl_b, rsl_b,
             cr_a, cl_a, cr_b, cl_b):
        my = lax.axis_index("i")
        left = lax.rem(my + N_DEV - 1, N_DEV)
        right = lax.rem(my + 1, N_DEV)

        barrier_sem = pltpu.get_barrier_semaphore()
        for nbr in (left, right):
            pl.semaphore_signal(
                barrier_sem, inc=1,
                device_id=(nbr,), device_id_type=pl.DeviceIdType.MESH,
            )
        pl.semaphore_wait(barrier_sem, 2)

        out_ref[...] = jnp.dot(
            x_ref[...].astype(jnp.bfloat16),
            w_ref[...].astype(jnp.bfloat16),
            preferred_element_type=jnp.float32,
        )

        def top(c):
            return pl.ds(c * CHUNK, HALF)

        def bot(c):
            return pl.ds(c * CHUNK + HALF, HALF)

        def mod(v):
            return lax.rem(v + 2 * N_DEV, N_DEV)

        def c_send_r(t):
            return mod(my - t) if t < RS_STEPS else mod(my + 1 - (t - RS_STEPS))

        def c_recv_r(t):
            return mod(my - t - 1) if t < RS_STEPS else mod(my - (t - RS_STEPS))

        def c_send_l(t):
            return mod(my + t) if t < RS_STEPS else mod(my - 1 + (t - RS_STEPS))

        def c_recv_l(t):
            return mod(my + t + 1) if t < RS_STEPS else mod(my + (t - RS_STEPS))

        def make_chain(col0, send_r, recv_r, send_l, recv_l,
                       ssr, rsr, ssl, rsl, cr, cl):
            cols = slice(col0, col0 + NCOL)
            pending = {}

            def send(t):
                slot = t % 2
                if t >= 2:
                    pl.semaphore_wait(cr, 1)
                    pl.semaphore_wait(cl, 1)
                send_r[slot] = out_ref[top(c_send_r(t)), cols].astype(jnp.bfloat16)
                send_l[slot] = out_ref[bot(c_send_l(t)), cols].astype(jnp.bfloat16)
                rr = pltpu.make_async_remote_copy(
                    src_ref=send_r.at[slot], dst_ref=recv_r.at[slot],
                    send_sem=ssr.at[slot], recv_sem=rsr.at[slot],
                    device_id=(right,), device_id_type=pl.DeviceIdType.MESH,
                )
                rl = pltpu.make_async_remote_copy(
                    src_ref=send_l.at[slot], dst_ref=recv_l.at[slot],
                    send_sem=ssl.at[slot], recv_sem=rsl.at[slot],
                    device_id=(left,), device_id_type=pl.DeviceIdType.MESH,
                )
                rr.start()
                rl.start()
                pending[t] = (rr, rl)

            def wait_consume(t):
                rr, rl = pending.pop(t)
                rr.wait()
                rl.wait()
                slot = t % 2
                if t < RS_STEPS:
                    out_ref[top(c_recv_r(t)), cols] += recv_r[slot].astype(jnp.float32)
                    out_ref[bot(c_recv_l(t)), cols] += recv_l[slot].astype(jnp.float32)
                    if t == RS_STEPS - 1:
                        c_or, c_ol = mod(my + 1), mod(my - 1)
                        out_ref[top(c_or), cols] = _gelu(out_ref[top(c_or), cols])
                        out_ref[bot(c_ol), cols] = _gelu(out_ref[bot(c_ol), cols])
                else:
                    out_ref[top(c_recv_r(t)), cols] = recv_r[slot].astype(jnp.float32)
                    out_ref[bot(c_recv_l(t)), cols] = recv_l[slot].astype(jnp.float32)
                if t + 2 < T_TOTAL:
                    pl.semaphore_signal(
                        cr, inc=1,
                        device_id=(left,), device_id_type=pl.DeviceIdType.MESH,
                    )
                    pl.semaphore_signal(
                        cl, inc=1,
                        device_id=(right,), device_id_type=pl.DeviceIdType.MESH,
                    )

            return send, wait_consume

        send_a, consume_a = make_chain(
            0, send_r_a, recv_r_a, send_l_a, recv_l_a,
            ssr_a, rsr_a, ssl_a, rsl_a, cr_a, cl_a)
        send_b, consume_b = make_chain(
            NCOL, send_r_b, recv_r_b, send_l_b, recv_l_b,
            ssr_b, rsr_b, ssl_b, rsl_b, cr_b, cl_b)

        send_a(0)
        send_b(0)
        for t in range(T_TOTAL):
            consume_a(t)
            if t + 1 < T_TOTAL:
                send_a(t + 1)
            consume_b(t)
            if t + 1 < T_TOTAL:
                send_b(t + 1)

    comm_buf = pltpu.VMEM((2, HALF, NCOL), jnp.bfloat16)
    return pl.pallas_call(
        body,
        out_shape=jax.ShapeDtypeStruct((M, N_OUT), jnp.float32),
        in_specs=[
            pl.BlockSpec(memory_space=pltpu.VMEM),
            pl.BlockSpec(memory_space=pltpu.VMEM),
        ],
        out_specs=pl.BlockSpec(memory_space=pltpu.VMEM),
        scratch_shapes=(
            [comm_buf] * 8
            + [pltpu.SemaphoreType.DMA((2,))] * 8
            + [pltpu.SemaphoreType.REGULAR] * 4
        ),
        compiler_params=pltpu.CompilerParams(
            collective_id=0,
            vmem_limit_bytes=100 * 1024 * 1024,
        ),
    )(x, w_mat)


# device time: 219853 ns/iter; 2.0288x vs baseline; 1.0577x over previous
import jax
import jax.numpy as jnp
from jax import lax
from jax.experimental import pallas as pl
from jax.experimental.pallas import tpu as pltpu

N_DEV = 16
M = 4096
N_OUT = 2048
CHUNK = M // N_DEV
HALF = CHUNK // 2
N_CHAINS = 4
NCOL = N_OUT // N_CHAINS

RS_STEPS = N_DEV - 1
T_TOTAL = 2 * (N_DEV - 1)

_GELU_C = 0.7978845608028654


def _gelu(y):
    return 0.5 * y * (1.0 + jnp.tanh(_GELU_C * (y + 0.044715 * y * y * y)))


def kernel(x, w_mat):
    def body(x_ref, w_ref, out_ref, *scratch):
        bufs = scratch[:4 * N_CHAINS]
        dma_sems = scratch[4 * N_CHAINS:8 * N_CHAINS]
        credits = scratch[8 * N_CHAINS:]

        my = lax.axis_index("i")
        left = lax.rem(my + N_DEV - 1, N_DEV)
        right = lax.rem(my + 1, N_DEV)

        barrier_sem = pltpu.get_barrier_semaphore()
        for nbr in (left, right):
            pl.semaphore_signal(
                barrier_sem, inc=1,
                device_id=(nbr,), device_id_type=pl.DeviceIdType.MESH,
            )
        pl.semaphore_wait(barrier_sem, 2)

        out_ref[...] = jnp.dot(
            x_ref[...].astype(jnp.bfloat16),
            w_ref[...].astype(jnp.bfloat16),
            preferred_element_type=jnp.float32,
        )

        def top(c):
            return pl.ds(c * CHUNK, HALF)

        def bot(c):
            return pl.ds(c * CHUNK + HALF, HALF)

        def mod(v):
            return lax.rem(v + 2 * N_DEV, N_DEV)

        def c_send_r(t):
            return mod(my - t) if t < RS_STEPS else mod(my + 1 - (t - RS_STEPS))

        def c_recv_r(t):
            return mod(my - t - 1) if t < RS_STEPS else mod(my - (t - RS_STEPS))

        def c_send_l(t):
            return mod(my + t) if t < RS_STEPS else mod(my - 1 + (t - RS_STEPS))

        def c_recv_l(t):
            return mod(my + t + 1) if t < RS_STEPS else mod(my + (t - RS_STEPS))

        def make_chain(k):
            send_r, recv_r, send_l, recv_l = bufs[4 * k:4 * k + 4]
            ssr, rsr, ssl, rsl = dma_sems[4 * k:4 * k + 4]
            cr, cl = credits[2 * k:2 * k + 2]
            cols = slice(k * NCOL, (k + 1) * NCOL)
            pending = {}

            def send(t):
                slot = t % 2
                if t >= 2:
                    pl.semaphore_wait(cr, 1)
                    pl.semaphore_wait(cl, 1)
                send_r[slot] = out_ref[top(c_send_r(t)), cols].astype(jnp.bfloat16)
                send_l[slot] = out_ref[bot(c_send_l(t)), cols].astype(jnp.bfloat16)
                rr = pltpu.make_async_remote_copy(
                    src_ref=send_r.at[slot], dst_ref=recv_r.at[slot],
                    send_sem=ssr.at[slot], recv_sem=rsr.at[slot],
                    device_id=(right,), device_id_type=pl.DeviceIdType.MESH,
                )
                rl = pltpu.make_async_remote_copy(
                    src_ref=send_l.at[slot], dst_ref=recv_l.at[slot],
                    send_sem=ssl.at[slot], recv_sem=rsl.at[slot],
                    device_id=(left,), device_id_type=pl.DeviceIdType.MESH,
                )
                rr.start()
                rl.start()
                pending[t] = (rr, rl)

            def wait_consume(t):
                rr, rl = pending.pop(t)
                rr.wait()
                rl.wait()
                slot = t % 2
                if t < RS_STEPS:
                    out_ref[top(c_recv_r(t)), cols] += recv_r[slot].astype(jnp.float32)
                    out_ref[bot(c_recv_l(t)), cols] += recv_l[slot].astype(jnp.float32)
                    if t == RS_STEPS - 1:
                        c_or, c_ol = mod(my + 1), mod(my - 1)
                        out_ref[top(c_or), cols] = _gelu(out_ref[top(c_or), cols])
                        out_ref[bot(c_ol), cols] = _gelu(out_ref[bot(c_ol), cols])
                else:
                    out_ref[top(c_recv_r(t)), cols] = recv_r[slot].astype(jnp.float32)
                    out_ref[bot(c_recv_l(t)), cols] = recv_l[slot].astype(jnp.float32)
                if t + 2 < T_TOTAL:
                    pl.semaphore_signal(
                        cr, inc=1,
                        device_id=(left,), device_id_type=pl.DeviceIdType.MESH,
                    )
                    pl.semaphore_signal(
                        cl, inc=1,
                        device_id=(right,), device_id_type=pl.DeviceIdType.MESH,
                    )

            return send, wait_consume

        chains = [make_chain(k) for k in range(N_CHAINS)]

        for send, _ in chains:
            send(0)
        for t in range(T_TOTAL):
            for send, wait_consume in chains:
                wait_consume(t)
                if t + 1 < T_TOTAL:
                    send(t + 1)

    comm_buf = pltpu.VMEM((2, HALF, NCOL), jnp.bfloat16)
    return pl.pallas_call(
        body,
        out_shape=jax.ShapeDtypeStruct((M, N_OUT), jnp.float32),
        in_specs=[
            pl.BlockSpec(memory_space=pltpu.VMEM),
            pl.BlockSpec(memory_space=pltpu.VMEM),
        ],
        out_specs=pl.BlockSpec(memory_space=pltpu.VMEM),
        scratch_shapes=(
            [comm_buf] * (4 * N_CHAINS)
            + [pltpu.SemaphoreType.DMA((2,))] * (4 * N_CHAINS)
            + [pltpu.SemaphoreType.REGULAR] * (2 * N_CHAINS)
        ),
        compiler_params=pltpu.CompilerParams(
            collective_id=0,
            vmem_limit_bytes=100 * 1024 * 1024,
        ),
    )(x, w_mat)
